# baseline (speedup 1.0000x reference)
import jax
import jax.numpy as jnp
from jax import lax
from jax.experimental import pallas as pl
from jax.experimental.pallas import tpu as pltpu
from jax.experimental.pallas import tpu_sc as plsc

B = 16384
L = 50
EMB = 64
HID = 128
NC = 2
NS = 16
NW = NC * NS
EPW = B // NW
CW = 128
NSUB = EPW // CW
NBUF = 4
NCH = L * NSUB


def _sc_pool_body(x4_hbm, table_hbm, out_hbm, idx_v, bufs, pooled_v, sems):
    wid = lax.axis_index("s") * NC + lax.axis_index("c")
    base = wid * EPW

    pltpu.sync_copy(x4_hbm.at[:, wid], idx_v)

    for k in range(NBUF - 1):
        pltpu.async_copy(
            table_hbm.at[idx_v.at[k // NSUB, k % NSUB]], bufs.at[k], sems.at[k]
        )

    def zero(r, _):
        for q in range(EMB // 16):
            pooled_v[r, pl.ds(q * 16, 16)] = jnp.zeros((16,), jnp.float32)
        return _

    lax.fori_loop(0, EPW, zero, None)

    def group(g, _):
        for par in range(NBUF):
            k = g * NBUF + par
            nxt = k + NBUF - 1

            @pl.when(nxt < NCH)
            def _():
                pltpu.async_copy(
                    table_hbm.at[idx_v.at[nxt // NSUB, (par + NBUF - 1) % NBUF]],
                    bufs.at[(par + NBUF - 1) % NBUF],
                    sems.at[(par + NBUF - 1) % NBUF],
                )

            pltpu.make_async_copy(
                table_hbm.at[idx_v.at[k // NSUB, par]], bufs.at[par], sems.at[par]
            ).wait()

            rowbase = par * CW

            def acc_row(j, _, par=par, rowbase=rowbase):
                for q in range(EMB // 16):
                    s = pl.ds(q * 16, 16)
                    pooled_v[rowbase + j, s] = (
                        pooled_v[rowbase + j, s] + bufs[par, j, s]
                    )
                return _

            lax.fori_loop(0, CW, acc_row, None)
        return _

    lax.fori_loop(0, NCH // NBUF, group, None)
    pltpu.sync_copy(pooled_v, out_hbm.at[pl.ds(base, EPW)])


def _sc_pool(x4, table):
    mesh = plsc.VectorSubcoreMesh(
        core_axis_name="c", subcore_axis_name="s", num_cores=NC, num_subcores=NS
    )
    return pl.kernel(
        _sc_pool_body,
        out_type=jax.ShapeDtypeStruct((B, EMB), jnp.float32),
        mesh=mesh,
        compiler_params=pltpu.CompilerParams(use_tc_tiling_on_sc=False),
        scratch_types=[
            pltpu.VMEM((L, NSUB, CW), jnp.int32),
            pltpu.VMEM((NBUF, CW, EMB), jnp.float32),
            pltpu.VMEM((EPW, EMB), jnp.float32),
            pltpu.SemaphoreType.DMA((NBUF,)),
        ],
    )(x4, table)


def _mlp_body(p_ref, w1_ref, b1_ref, w2_ref, b2_ref, o_ref):
    p = p_ref[...] * (1.0 / L)
    h = lax.dot_general(
        p, w1_ref[...], (((1,), (1,)), ((), ())),
        preferred_element_type=jnp.float32,
    )
    h = jnp.maximum(h + b1_ref[...], 0.0)
    o = lax.dot_general(
        h, w2_ref[...], (((1,), (1,)), ((), ())),
        preferred_element_type=jnp.float32,
    )
    o_ref[...] = o + b2_ref[...]


def _mlp(pooled, W1, b1, W2p, b2p):
    BLK = 2048
    return pl.pallas_call(
        _mlp_body,
        grid=(B // BLK,),
        in_specs=[
            pl.BlockSpec((BLK, EMB), lambda i: (i, 0)),
            pl.BlockSpec((HID, EMB), lambda i: (0, 0)),
            pl.BlockSpec((1, HID), lambda i: (0, 0)),
            pl.BlockSpec((8, HID), lambda i: (0, 0)),
            pl.BlockSpec((1, 8), lambda i: (0, 0)),
        ],
        out_specs=pl.BlockSpec((BLK, 8), lambda i: (i, 0)),
        out_shape=jax.ShapeDtypeStruct((B, 8), jnp.float32),
    )(pooled, W1, b1, W2p, b2p)


def kernel(x, table, W1, b1, W2, b2):
    x4 = jnp.transpose(x).reshape(L, NW, NSUB, CW)
    pooled = _sc_pool(x4, table)
    W2p = jnp.pad(W2, ((0, 7), (0, 0)))
    b2p = jnp.pad(b2, (0, 7)).reshape(1, 8)
    out8 = _mlp(pooled, W1, b1.reshape(1, HID), W2p, b2p)
    return out8[:, :1]
